# Initial kernel scaffold; baseline (speedup 1.0000x reference)
import jax
import jax.numpy as jnp
from jax.experimental import pallas as pl


def kernel(inputs, send_edges, recv_edges, edge2node_mat):
    raise NotImplementedError("write your pallas kernel here")


# speedup vs baseline: 107.2472x; 107.2472x over previous
import jax
import jax.numpy as jnp
from jax.experimental import pallas as pl
from jax.experimental.pallas import tpu as pltpu


def _build_m_kernel(send_ref, recv_ref, m_ref):
    n2 = m_ref.shape[0]
    n = n2 // 2
    e = send_ref.shape[1]
    ids = jax.lax.broadcasted_iota(jnp.int32, (n, e), 0)
    st = (ids == send_ref[...]).astype(jnp.float32)
    rt = (ids == recv_ref[...]).astype(jnp.float32)
    contract = (((1,), (1,)), ((), ()))
    m1 = jax.lax.dot_general(rt, st, contract, preferred_element_type=jnp.float32)
    m2 = jax.lax.dot_general(rt, rt, contract, preferred_element_type=jnp.float32)
    inv = 1.0 / (n - 1)
    m_ref[...] = jnp.concatenate([m1, m2], axis=0) * inv


def _apply_kernel(x_ref, m_ref, out_ref):
    n = x_ref.shape[1]
    d = x_ref.shape[2]
    m = m_ref[...]
    for j in range(x_ref.shape[0]):
        y = jnp.dot(m, x_ref[j], preferred_element_type=jnp.float32)
        out_ref[j, :, 0:d] = y[:n]
        out_ref[j, :, d : 2 * d] = y[n:]


def kernel(inputs, send_edges, recv_edges, edge2node_mat):
    b, n, d = inputs.shape
    e = send_edges.shape[0]
    send2d = send_edges.reshape(1, e).astype(jnp.int32)
    recv2d = recv_edges.reshape(1, e).astype(jnp.int32)

    m = pl.pallas_call(
        _build_m_kernel,
        out_shape=jax.ShapeDtypeStruct((2 * n, n), jnp.float32),
    )(send2d, recv2d)

    bb = 8
    out = pl.pallas_call(
        _apply_kernel,
        grid=(b // bb,),
        in_specs=[
            pl.BlockSpec((bb, n, d), lambda i: (i, 0, 0)),
            pl.BlockSpec((2 * n, n), lambda i: (0, 0)),
        ],
        out_specs=pl.BlockSpec((bb, n, 2 * d), lambda i: (i, 0, 0)),
        out_shape=jax.ShapeDtypeStruct((b, n, 2 * d), jnp.float32),
        compiler_params=pltpu.CompilerParams(
            dimension_semantics=("parallel",),
        ),
    )(inputs, m)
    return out
